# baseline (speedup 1.0000x reference)
import functools

import jax
import jax.numpy as jnp
from jax.experimental import pallas as pl
from jax.experimental.pallas import tpu as pltpu

_VMEM_LIMIT = 62 * 1024 * 1024


def _go_prep_kernel(se, ge, mW1, mb1, mW2, mb2, g1W, sW1, sb1, sW2, sb2,
                    hsem_out, sup1_out, seqo_out):
    f32 = jnp.float32

    @pl.when(pl.program_id(0) == 0)
    def _seq():
        s = jnp.maximum(jnp.dot(se[...], sW1[...], preferred_element_type=f32) + sb1[...], 0.0)
        seqo_out[...] = jnp.dot(s, sW2[...], preferred_element_type=f32) + sb2[...]

    geb = ge[...]
    h = jnp.maximum(jnp.dot(geb, mW1[...], preferred_element_type=f32) + mb1[...], 0.0)
    hsem_out[...] = jnp.dot(h, mW2[...], preferred_element_type=f32) + mb2[...]
    sup1_out[...] = jnp.dot(geb, g1W[...], preferred_element_type=f32)


def _spmm_kernel(s1_any, a_any, g1b, g2W, g2b, hstr_out,
                 abuf, s1_scr, sup2_scr, asems, s1_sem, *, n_p1, bm):
    g = pl.program_id(0)
    f32 = jnp.float32

    def a_copy(step, slot):
        tile = jnp.where(step < n_p1, step, step - n_p1)
        return pltpu.make_async_copy(
            a_any.at[pl.ds(tile * bm, bm), :], abuf.at[slot], asems.at[slot])

    cur = jax.lax.rem(g, 2)
    nxt = jax.lax.rem(g + 1, 2)

    @pl.when(g == 0)
    def _warmup():
        pltpu.make_async_copy(s1_any, s1_scr, s1_sem).start()
        a_copy(0, 0).start()
        pltpu.make_async_copy(s1_any, s1_scr, s1_sem).wait()

    @pl.when(g < 2 * n_p1 - 1)
    def _prefetch():
        a_copy(g + 1, nxt).start()

    a_copy(g, cur).wait()
    a = abuf[cur]

    @pl.when(g < n_p1)
    def _phase1():
        x = jnp.maximum(
            jax.lax.dot_general(a, s1_scr[...], (((1,), (0,)), ((), ())),
                                preferred_element_type=f32) + g1b[...], 0.0)
        sup2_scr[pl.ds(jnp.minimum(g, n_p1 - 1) * bm, bm), :] = jnp.dot(
            x, g2W[...], preferred_element_type=f32)

    @pl.when(g >= n_p1)
    def _phase2():
        hstr_out[...] = jnp.maximum(
            jax.lax.dot_general(a, sup2_scr[...], (((1,), (0,)), ((), ())),
                                preferred_element_type=f32) + g2b[...], 0.0)


def _pred_kernel(seqo, hsem, hstr, pred_out, *, nh1):
    f32 = jnp.float32
    lo = jax.lax.dot_general(seqo[:, :nh1], hsem[...], (((1,), (1,)), ((), ())),
                             preferred_element_type=f32)
    hi = jax.lax.dot_general(seqo[:, nh1:], hstr[...], (((1,), (1,)), ((), ())),
                             preferred_element_type=f32)
    pred_out[...] = jax.nn.sigmoid(lo + hi)


def kernel(sequence_embedding, go_embedding, adjacency_matrix,
           mlp_W1, mlp_b1, mlp_W2, mlp_b2,
           gc1_W, gc1_b, gc2_W, gc2_b,
           seq_W1, seq_b1, seq_W2, seq_b2):
    n_go, go_feat = go_embedding.shape
    b, seq_feat = sequence_embedding.shape
    nh0 = mlp_W1.shape[1]
    nh1 = mlp_W2.shape[1]
    f32 = jnp.float32

    mb1 = mlp_b1.reshape(1, -1)
    mb2 = mlp_b2.reshape(1, -1)
    g1b = gc1_b.reshape(1, -1)
    g2b = gc2_b.reshape(1, -1)
    sb1 = seq_b1.reshape(1, -1)
    sb2 = seq_b2.reshape(1, -1)

    full = lambda shape: pl.BlockSpec(shape, lambda m: (0, 0))
    tiled = lambda bm, n: pl.BlockSpec((bm, n), lambda m: (m, 0))
    anyspace = pl.BlockSpec(memory_space=pl.ANY)

    BG = 2000
    h_semantic, support1, seq_output = pl.pallas_call(
        _go_prep_kernel,
        grid=(n_go // BG,),
        in_specs=[full((b, seq_feat)), tiled(BG, go_feat),
                  full(mlp_W1.shape), full(mb1.shape), full(mlp_W2.shape),
                  full(mb2.shape), full(gc1_W.shape),
                  full(seq_W1.shape), full(sb1.shape), full(seq_W2.shape),
                  full(sb2.shape)],
        out_specs=[tiled(BG, nh1), tiled(BG, nh0), full((b, 2 * nh1))],
        out_shape=[jax.ShapeDtypeStruct((n_go, nh1), f32),
                   jax.ShapeDtypeStruct((n_go, nh0), f32),
                   jax.ShapeDtypeStruct((b, 2 * nh1), f32)],
        compiler_params=pltpu.CompilerParams(
            dimension_semantics=("parallel",), vmem_limit_bytes=_VMEM_LIMIT),
    )(sequence_embedding, go_embedding, mlp_W1, mb1, mlp_W2, mb2, gc1_W,
      seq_W1, sb1, seq_W2, sb2)

    BM = 400
    n_p1 = n_go // BM
    hstr_idx = lambda g: (jnp.maximum(g - n_p1, 0), 0)
    h_structure = pl.pallas_call(
        functools.partial(_spmm_kernel, n_p1=n_p1, bm=BM),
        grid=(2 * n_p1,),
        in_specs=[anyspace, anyspace, full(g1b.shape), full(gc2_W.shape),
                  full(g2b.shape)],
        out_specs=pl.BlockSpec((BM, nh1), hstr_idx),
        out_shape=jax.ShapeDtypeStruct((n_go, nh1), f32),
        scratch_shapes=[pltpu.VMEM((2, BM, n_go), f32),
                        pltpu.VMEM((n_go, nh0), f32),
                        pltpu.VMEM((n_go, nh1), f32),
                        pltpu.SemaphoreType.DMA((2,)),
                        pltpu.SemaphoreType.DMA],
        compiler_params=pltpu.CompilerParams(vmem_limit_bytes=_VMEM_LIMIT),
    )(support1, adjacency_matrix, g1b, gc2_W, g2b)

    BB = 256
    prediction = pl.pallas_call(
        functools.partial(_pred_kernel, nh1=nh1),
        grid=(b // BB,),
        in_specs=[tiled(BB, 2 * nh1), full((n_go, nh1)), full((n_go, nh1))],
        out_specs=tiled(BB, n_go),
        out_shape=jax.ShapeDtypeStruct((b, n_go), f32),
        compiler_params=pltpu.CompilerParams(
            dimension_semantics=("parallel",), vmem_limit_bytes=_VMEM_LIMIT),
    )(seq_output, h_semantic, h_structure)

    return (h_semantic, h_structure, prediction)
